# baseline (speedup 1.0000x reference)
import jax
import jax.numpy as jnp
from jax.experimental import pallas as pl
from jax.experimental.pallas import tpu as pltpu

_D = 768
_N0, _N1, _N2 = 1024, 256, 64
_NTOT = _N0 + _N1 + _N2
_B = 16
_CB = 2
_NC = _B // _CB
_NS = 3


def _body(f0, f1, f2, se, pe, out, buf, tbl, se_v, in_sems, out_sems,
          tbl_sem):
    pltpu.make_async_copy(pe.at[0, pl.ds(0, _N0)],
                          tbl.at[pl.ds(0, _N0)], tbl_sem).start()
    pltpu.make_async_copy(pe.at[1, pl.ds(0, _N1)],
                          tbl.at[pl.ds(_N0, _N1)], tbl_sem).start()
    pltpu.make_async_copy(pe.at[2, pl.ds(0, _N2)],
                          tbl.at[pl.ds(_N0 + _N1, _N2)], tbl_sem).start()
    pltpu.make_async_copy(se, se_v, tbl_sem).start()
    pltpu.make_async_copy(pe.at[0, pl.ds(0, _N0)],
                          tbl.at[pl.ds(0, _N0)], tbl_sem).wait()
    pltpu.make_async_copy(pe.at[1, pl.ds(0, _N1)],
                          tbl.at[pl.ds(_N0, _N1)], tbl_sem).wait()
    pltpu.make_async_copy(pe.at[2, pl.ds(0, _N2)],
                          tbl.at[pl.ds(_N0 + _N1, _N2)], tbl_sem).wait()
    pltpu.make_async_copy(se, se_v, tbl_sem).wait()
    tbl[pl.ds(0, _N0), :] = tbl[pl.ds(0, _N0), :] + se_v[0, :][None, :]
    tbl[pl.ds(_N0, _N1), :] = tbl[pl.ds(_N0, _N1), :] + se_v[1, :][None, :]
    tbl[pl.ds(_N0 + _N1, _N2), :] = (
        tbl[pl.ds(_N0 + _N1, _N2), :] + se_v[2, :][None, :])

    def in_copies(c, s):
        return (
            pltpu.make_async_copy(f0.at[pl.ds(c * _CB, _CB)],
                                  buf.at[s, :, pl.ds(0, _N0)],
                                  in_sems.at[s]),
            pltpu.make_async_copy(f1.at[pl.ds(c * _CB, _CB)],
                                  buf.at[s, :, pl.ds(_N0, _N1)],
                                  in_sems.at[s]),
            pltpu.make_async_copy(f2.at[pl.ds(c * _CB, _CB)],
                                  buf.at[s, :, pl.ds(_N0 + _N1, _N2)],
                                  in_sems.at[s]),
        )

    def out_copy(c, s):
        return pltpu.make_async_copy(buf.at[s],
                                     out.at[pl.ds(c * _CB, _CB)],
                                     out_sems.at[s])

    def start_in(c):
        for h in in_copies(c, c % _NS):
            h.start()

    start_in(0)
    start_in(1)
    for c in range(_NC):
        s = c % _NS
        for h in in_copies(c, s):
            h.wait()
        buf[s] = buf[s] + tbl[...][None, :, :]
        out_copy(c, s).start()
        if c + 2 < _NC:
            if c >= 1:
                out_copy(c - 1, (c - 1) % _NS).wait()
            start_in(c + 2)
    for c in range(_NC - 3, _NC):
        out_copy(c, c % _NS).wait()


def kernel(features_per_scale_0, features_per_scale_1, features_per_scale_2,
           scale_embeddings, patch_embeddings):
    any_spec = pl.BlockSpec(memory_space=pltpu.MemorySpace.HBM)
    return pl.pallas_call(
        _body,
        in_specs=[any_spec] * 5,
        out_specs=any_spec,
        out_shape=jax.ShapeDtypeStruct((_B, _NTOT, _D), jnp.float32),
        scratch_shapes=[
            pltpu.VMEM((_NS, _CB, _NTOT, _D), jnp.float32),
            pltpu.VMEM((_NTOT, _D), jnp.float32),
            pltpu.VMEM((3, _D), jnp.float32),
            pltpu.SemaphoreType.DMA((_NS,)),
            pltpu.SemaphoreType.DMA((_NS,)),
            pltpu.SemaphoreType.DMA,
        ],
        compiler_params=pltpu.CompilerParams(
            vmem_limit_bytes=128 * 1024 * 1024),
    )(features_per_scale_0, features_per_scale_1, features_per_scale_2,
      scale_embeddings, patch_embeddings)


# speedup vs baseline: 1.0073x; 1.0073x over previous
import jax
import jax.numpy as jnp
from jax.experimental import pallas as pl
from jax.experimental.pallas import tpu as pltpu

_D = 768
_N0, _N1, _N2 = 1024, 256, 64
_NTOT = _N0 + _N1 + _N2


_BB = 2


def _body(f0_ref, f1_ref, f2_ref, se_ref, pe0_ref, pe1_ref, pe2_ref, out_ref):
    out_ref[:, 0:_N0, :] = (
        f0_ref[...] + (se_ref[0, :][None, None, :] + pe0_ref[...]))
    out_ref[:, _N0:_N0 + _N1, :] = (
        f1_ref[...] + (se_ref[1, :][None, None, :] + pe1_ref[...]))
    out_ref[:, _N0 + _N1:_NTOT, :] = (
        f2_ref[...] + (se_ref[2, :][None, None, :] + pe2_ref[...]))


def kernel(features_per_scale_0, features_per_scale_1, features_per_scale_2,
           scale_embeddings, patch_embeddings):
    B = features_per_scale_0.shape[0]

    return pl.pallas_call(
        _body,
        grid=(B // _BB,),
        in_specs=[
            pl.BlockSpec((_BB, _N0, _D), lambda b: (b, 0, 0)),
            pl.BlockSpec((_BB, _N1, _D), lambda b: (b, 0, 0)),
            pl.BlockSpec((_BB, _N2, _D), lambda b: (b, 0, 0)),
            pl.BlockSpec((3, _D), lambda b: (0, 0)),
            pl.BlockSpec((1, _N0, _D), lambda b: (0, 0, 0)),
            pl.BlockSpec((1, _N1, _D), lambda b: (1, 0, 0)),
            pl.BlockSpec((1, _N2, _D), lambda b: (2, 0, 0)),
        ],
        out_specs=pl.BlockSpec((_BB, _NTOT, _D), lambda b: (b, 0, 0)),
        out_shape=jax.ShapeDtypeStruct((B, _NTOT, _D), jnp.float32),
        compiler_params=pltpu.CompilerParams(
            dimension_semantics=("arbitrary",),
            vmem_limit_bytes=120 * 1024 * 1024),
    )(features_per_scale_0, features_per_scale_1, features_per_scale_2,
      scale_embeddings, patch_embeddings, patch_embeddings, patch_embeddings)
